# Initial kernel scaffold; baseline (speedup 1.0000x reference)
import jax
import jax.numpy as jnp
from jax.experimental import pallas as pl


def kernel(input_data, adj_list, W0, b0, W1, b1, W2, b2, Wm, bm, W_ih, W_hh, b_ih, b_hh, Wo, bo):
    raise NotImplementedError("write your pallas kernel here")


# speedup vs baseline: 5.2834x; 5.2834x over previous
import functools

import jax
import jax.numpy as jnp
from jax.experimental import pallas as pl
from jax.experimental.pallas import tpu as pltpu

_HID = 32
_INIT_LEN = 4
_F32 = jnp.float32
_BF = jnp.bfloat16


def _step_kernel(a_ref, x_ref, xd_ref, wk0_ref, b0_ref, wk12_ref, b1_ref,
                 b2_ref, wkm_ref, bm_ref,
                 wgi_ref, bgi_ref, wgh_ref, bgh_ref, wok_ref, bok_ref,
                 out_ref, hid_ref):
    i = pl.program_id(0)
    A = a_ref[0]
    C = A.shape[0]
    Ab = A.astype(_BF)

    d_r = jnp.sum(A, axis=1, keepdims=True) + 1.0
    d_c = jnp.sum(A, axis=0, keepdims=True).T + 1.0
    dinv_r = jax.lax.rsqrt(d_r)
    dinv_c = jax.lax.rsqrt(d_c)

    def norm_matmul_f(Z):
        Zs = dinv_r * Z
        AZ = jnp.dot(Ab, Zs.astype(_BF), preferred_element_type=_F32)
        return dinv_r * (AZ + Zs)

    def norm_matmul_b(Z):
        Zs = dinv_c * Z
        At_Zs = jax.lax.dot_general(Ab, Zs.astype(_BF),
                                    (((0,), (0,)), ((), ())),
                                    preferred_element_type=_F32)
        return dinv_c * (At_Zs + Zs)

    @pl.when(i == 0)
    def _init():
        x0 = x_ref[0]
        y0 = jnp.dot(x0, wk0_ref[...], preferred_element_type=_F32)
        hid_ref[...] = jax.nn.relu(norm_matmul_f(y0) + b0_ref[...])

    h = hid_ref[...]
    G = h.shape[1]

    y = jnp.dot(h.astype(_BF), wk12_ref[...],
                preferred_element_type=_F32)
    fh = jax.nn.relu(norm_matmul_f(y[:, :G]) + b1_ref[...])
    bh = jax.nn.relu(norm_matmul_b(y[:, G:]) + b2_ref[...])

    hs = (jnp.dot(jnp.concatenate([fh, bh], axis=1).astype(_BF), wkm_ref[...],
                  preferred_element_type=_F32)
          + bm_ref[...])

    x_t = x_ref[i]
    gi = jnp.dot(x_t, wgi_ref[...], preferred_element_type=_F32) + bgi_ref[...]
    gh = jnp.dot(hs.astype(_BF), wgh_ref[...],
                 preferred_element_type=_F32) + bgh_ref[...]

    r = jax.nn.sigmoid(gi[:, :G] + gh[:, :G])
    z = jax.nn.sigmoid(gi[:, G:2 * G] + gh[:, G:2 * G])
    n = jnp.tanh(gi[:, 2 * G:] + r * gh[:, 2 * G:])
    h_new = (1.0 - z) * n + z * hs
    hid_ref[...] = h_new

    @pl.when(i >= _INIT_LEN)
    def _emit():
        half = C // 2
        pred = (jnp.dot(h_new[half:].astype(_BF), wok_ref[...],
                        preferred_element_type=_F32)
                + bok_ref[...])
        obs = xd_ref[i + 1, :half, :]
        out_ref[i - _INIT_LEN] = jnp.concatenate([obs, pred], axis=0)


@functools.partial(jax.jit, static_argnums=())
def kernel(input_data, adj_list, W0, b0, W1, b1, W2, b2, Wm, bm,
           W_ih, W_hh, b_ih, b_hh, Wo, bo):
    B, T, C, F = input_data.shape
    HID = _HID
    G = B * HID

    xT = jnp.transpose(input_data, (1, 2, 0, 3)).reshape(T, C, B * F).astype(_BF)
    xd = jnp.transpose(input_data[..., 0], (1, 2, 0))

    eyeB = jnp.eye(B, dtype=_F32)

    def kron_w(W):
        return jnp.kron(eyeB, W).astype(_BF)

    def tile_b(v):
        return jnp.tile(v, B)[None, :]

    Wk0 = kron_w(W0)
    Wk12 = jnp.concatenate([kron_w(W1), kron_w(W2)], axis=1)
    Wkm = jnp.concatenate([kron_w(Wm[:HID]), kron_w(Wm[HID:])], axis=0)
    WihT = W_ih.T
    WhhT = W_hh.T
    Wgi = jnp.concatenate(
        [kron_w(WihT[:, g * HID:(g + 1) * HID]) for g in range(3)], axis=1)
    Wgh = jnp.concatenate(
        [kron_w(WhhT[:, g * HID:(g + 1) * HID]) for g in range(3)], axis=1)
    bgi = jnp.concatenate(
        [tile_b(b_ih[g * HID:(g + 1) * HID]) for g in range(3)], axis=1)
    bgh = jnp.concatenate(
        [tile_b(b_hh[g * HID:(g + 1) * HID]) for g in range(3)], axis=1)
    Wok = kron_w(Wo)
    bok = jnp.tile(bo, B)[None, :]

    num_steps = T - 1
    num_out = num_steps - _INIT_LEN

    full = lambda shape: pl.BlockSpec(shape, lambda i: (0,) * len(shape))

    out = pl.pallas_call(
        _step_kernel,
        grid=(num_steps,),
        in_specs=[
            pl.BlockSpec((1, C, C), lambda i: (i, 0, 0)),
            full((T, C, B * F)),
            full((T, C, B)),
            full(Wk0.shape), full((1, G)),
            full(Wk12.shape), full((1, G)), full((1, G)),
            full(Wkm.shape), full((1, G)),
            full(Wgi.shape), full((1, 3 * G)),
            full(Wgh.shape), full((1, 3 * G)),
            full(Wok.shape), full((1, B)),
        ],
        out_specs=full((num_out, C, B)),
        out_shape=jax.ShapeDtypeStruct((num_out, C, B), _F32),
        scratch_shapes=[pltpu.VMEM((C, G), _F32)],
        compiler_params=pltpu.CompilerParams(
            dimension_semantics=("arbitrary",)),
    )(adj_list, xT, xd,
      Wk0, tile_b(b0), Wk12, tile_b(b1), tile_b(b2),
      Wkm, tile_b(bm), Wgi, bgi, Wgh, bgh, Wok, bok)

    return jnp.transpose(out, (2, 0, 1))[..., None]
